# baseline (speedup 1.0000x reference)
import functools

import jax
import jax.numpy as jnp
from jax import lax
from jax.experimental import pallas as pl
from jax.experimental.pallas import tpu as pltpu
from jax.experimental.pallas import tpu_sc as plsc

N = 10000
E = 320000
R = 8
B = 4
D = 128

NC = 2
NS = 16
NW = NC * NS
C = 80
ROWS_PER_W = E // C // NW
NPASS = 5
PR = ROWS_PER_W // NPASS
NPAD = 10240
RPT = NPAD // NS
ZC = 64
CW = D

TN = 2000


def _dense_body(comp_ref, x_ref, basis_ref, root_ref, h_ref, xr_ref):
    x = x_ref[...]
    for r in range(R):
        w = comp_ref[r, 0] * basis_ref[0]
        for b in range(1, B):
            w = w + comp_ref[r, b] * basis_ref[b]
        h_ref[r] = jnp.dot(x, w, preferred_element_type=jnp.float32)
    xr_ref[...] = jnp.dot(x, root_ref[...], preferred_element_type=jnp.float32)


_dense = pl.pallas_call(
    _dense_body,
    grid=(N // TN,),
    in_specs=[
        pl.BlockSpec(memory_space=pltpu.SMEM),
        pl.BlockSpec((TN, D), lambda i: (i, 0)),
        pl.BlockSpec((B, D, D), lambda i: (0, 0, 0)),
        pl.BlockSpec((D, D), lambda i: (0, 0)),
    ],
    out_specs=[
        pl.BlockSpec((R, TN, D), lambda i: (0, i, 0)),
        pl.BlockSpec((TN, D), lambda i: (i, 0)),
    ],
    out_shape=[
        jax.ShapeDtypeStruct((R, N, D), jnp.float32),
        jax.ShapeDtypeStruct((N, D), jnp.float32),
    ],
)


def _combine_body(p_ref, cnt_ref, xr_ref, bias_ref, o_ref):
    agg = p_ref[0] + p_ref[1]
    deg = cnt_ref[0, :, 0:1] + cnt_ref[1, :, 0:1]
    inv = 1.0 / jnp.maximum(deg, 1.0)
    v = agg * inv + xr_ref[...] + bias_ref[...]
    o_ref[...] = jnp.where(v > 0, v, jnp.exp(jnp.minimum(v, 0.0)) - 1.0)


_combine = pl.pallas_call(
    _combine_body,
    grid=(N // TN,),
    in_specs=[
        pl.BlockSpec((NC, TN, D), lambda i: (0, i, 0)),
        pl.BlockSpec((NC, TN, CW), lambda i: (0, i, 0)),
        pl.BlockSpec((TN, D), lambda i: (i, 0)),
        pl.BlockSpec((1, D), lambda i: (0, 0)),
    ],
    out_specs=pl.BlockSpec((TN, D), lambda i: (i, 0)),
    out_shape=jax.ShapeDtypeStruct((N, D), jnp.float32),
)


_mesh = plsc.VectorSubcoreMesh(core_axis_name="c", subcore_axis_name="s")


@functools.partial(
    pl.kernel,
    out_type=jax.ShapeDtypeStruct((NC, NPAD, D), jnp.float32),
    mesh=_mesh,
    scratch_types=[
        pltpu.VMEM((PR, C), jnp.int32),
        pltpu.VMEM((PR, C), jnp.int32),
        pltpu.VMEM((C, D), jnp.float32),
        pltpu.VMEM((C, D), jnp.float32),
        pltpu.VMEM((C, D), jnp.float32),
        pltpu.VMEM((C, D), jnp.float32),
        pltpu.VMEM_SHARED((NPAD, D), jnp.float32),
        pltpu.SemaphoreType.DMA,
        pltpu.SemaphoreType.DMA,
        pltpu.SemaphoreType.DMA,
        pltpu.SemaphoreType.DMA,
        pltpu.SemaphoreType.DMA,
        pltpu.SemaphoreType.DMA,
        pltpu.SemaphoreType.DMA,
        pltpu.SemaphoreType.DMA,
        pltpu.SemaphoreType.DMA,
        pltpu.SemaphoreType.DMA,
    ],
)
def _sc_agg(h_hbm, gidx_hbm, didx_hbm, out_hbm,
            gbuf, dbuf, rows0, rows1, rows2, rows3, acc,
            sg0, sg1, sg2, sg3, ss0, ss1, ss2, ss3, si0, si1):
    cid = lax.axis_index("c")
    sid = lax.axis_index("s")
    wid = sid * NC + cid

    bufs = (rows0, rows1, rows2, rows3)
    sgs = (sg0, sg1, sg2, sg3)
    sss = (ss0, ss1, ss2, ss3)

    pltpu.async_copy(gidx_hbm.at[wid, 0], gbuf, si0)
    pltpu.async_copy(didx_hbm.at[wid, 0], dbuf, si1)

    def zrow(i, carry):
        for cc in range(D // 16):
            rows0[i, pl.ds(cc * 16, 16)] = jnp.zeros((16,), jnp.float32)
        return carry

    lax.fori_loop(0, ZC, zrow, 0)
    for k in range(RPT // ZC):
        pltpu.sync_copy(rows0.at[pl.ds(0, ZC)],
                        acc.at[pl.ds(sid * RPT + k * ZC, ZC)])

    pltpu.make_async_copy(gidx_hbm.at[wid, 0], gbuf, si0).wait()
    pltpu.make_async_copy(didx_hbm.at[wid, 0], dbuf, si1).wait()
    plsc.subcore_barrier()

    def g_start(j, b):
        pltpu.async_copy(h_hbm.at[gbuf.at[j]], bufs[b], sgs[b])

    def g_wait(j, b):
        pltpu.make_async_copy(h_hbm.at[gbuf.at[j]], bufs[b], sgs[b]).wait()

    def s_start(j, b):
        pltpu.async_copy(bufs[b], acc.at[dbuf.at[j]], sss[b], add=True)

    def s_wait(j, b):
        pltpu.make_async_copy(bufs[b], acc.at[dbuf.at[j]], sss[b]).wait()

    for p in range(NPASS):
        if p > 0:
            pltpu.sync_copy(gidx_hbm.at[wid, p], gbuf)
            pltpu.sync_copy(didx_hbm.at[wid, p], dbuf)
        g_start(0, 0)
        g_start(1, 1)
        g_start(2, 2)
        g_wait(0, 0)
        s_start(0, 0)
        g_start(3, 3)
        g_wait(1, 1)
        s_start(1, 1)
        s_wait(0, 0)
        g_start(4, 0)
        g_wait(2, 2)
        s_start(2, 2)
        s_wait(1, 1)
        g_start(5, 1)

        def block(m, carry):
            j = 4 * m + 3
            for i in range(4):
                b = (3 + i) % 4
                g_wait(j + i, b)
                s_start(j + i, b)
                s_wait(j + i - 1, (b + 3) % 4)
                g_start(j + i + 3, (b + 3) % 4)
            return carry

        lax.fori_loop(0, (PR - 9) // 4, block, 0)
        for j in range(PR - 6, PR):
            b = j % 4
            g_wait(j, b)
            s_start(j, b)
            s_wait(j - 1, (b + 3) % 4)
            if j + 3 < PR:
                g_start(j + 3, (j + 3) % 4)
        s_wait(PR - 1, (PR - 1) % 4)

    plsc.subcore_barrier()
    pltpu.sync_copy(acc.at[pl.ds(sid * RPT, RPT)],
                    out_hbm.at[cid, pl.ds(sid * RPT, RPT)])


@functools.partial(
    pl.kernel,
    out_type=jax.ShapeDtypeStruct((NC, NPAD, CW), jnp.float32),
    mesh=_mesh,
    scratch_types=[
        pltpu.VMEM((PR, C), jnp.int32),
        pltpu.VMEM((C, CW), jnp.float32),
        pltpu.VMEM_SHARED((NPAD, CW), jnp.float32),
        pltpu.SemaphoreType.DMA,
        pltpu.SemaphoreType.DMA,
    ],
)
def _sc_deg(didx_hbm, out_hbm, dbuf, ones, cnt, sem0, sem1):
    cid = lax.axis_index("c")
    sid = lax.axis_index("s")
    wid = sid * NC + cid

    sems = (sem0, sem1)

    def zrow(i, carry):
        for cc in range(CW // 16):
            ones[i, pl.ds(cc * 16, 16)] = jnp.zeros((16,), jnp.float32)
        return carry

    lax.fori_loop(0, ZC, zrow, 0)
    for k in range(RPT // ZC):
        pltpu.sync_copy(ones.at[pl.ds(0, ZC)],
                        cnt.at[pl.ds(sid * RPT + k * ZC, ZC)])

    def orow(i, carry):
        for cc in range(CW // 16):
            ones[i, pl.ds(cc * 16, 16)] = jnp.ones((16,), jnp.float32)
        return carry

    lax.fori_loop(0, C, orow, 0)
    plsc.subcore_barrier()

    def s_start(j, b):
        pltpu.async_copy(ones, cnt.at[dbuf.at[j]], sems[b], add=True)

    def s_wait(j, b):
        pltpu.make_async_copy(ones, cnt.at[dbuf.at[j]], sems[b]).wait()

    for p in range(NPASS):
        pltpu.sync_copy(didx_hbm.at[wid, p], dbuf)
        s_start(0, 0)
        s_start(1, 1)

        def step(m, carry):
            j = 2 * m + 2
            s_wait(j - 2, 0)
            s_start(j, 0)
            s_wait(j - 1, 1)
            s_start(j + 1, 1)
            return carry

        lax.fori_loop(0, (PR - 3) // 2, step, 0)
        jl = PR - 1
        s_wait(jl - 2, (jl - 2) % 2)
        s_start(jl, jl % 2)
        s_wait(jl - 1, (jl - 1) % 2)
        s_wait(jl, jl % 2)
    plsc.subcore_barrier()
    pltpu.sync_copy(cnt.at[pl.ds(sid * RPT, RPT)],
                    out_hbm.at[cid, pl.ds(sid * RPT, RPT)])


def kernel(x, edge_index, edge_attr,
           basis0, comp0, root0, bias0,
           basis1, comp1, root1, bias1,
           basis2, comp2, root2, bias2,
           basis3, comp3, root3, bias3):
    src = edge_index[0]
    dst = edge_index[1]
    gidx = (edge_attr * N + src).astype(jnp.int32).reshape(NW, NPASS, PR, C)
    didx = dst.astype(jnp.int32).reshape(NW, NPASS, PR, C)

    cnt = _sc_deg(didx)

    h = x
    for basis, comp, root, bias in (
        (basis0, comp0, root0, bias0),
        (basis1, comp1, root1, bias1),
        (basis2, comp2, root2, bias2),
        (basis3, comp3, root3, bias3),
    ):
        hr, xr = _dense(comp, h, basis, root)
        part = _sc_agg(hr.reshape(R * N, D), gidx, didx)
        h = _combine(part, cnt, xr, bias.reshape(1, D))
    return h


# speedup vs baseline: 1.0224x; 1.0224x over previous
import functools

import jax
import jax.numpy as jnp
from jax import lax
from jax.experimental import pallas as pl
from jax.experimental.pallas import tpu as pltpu
from jax.experimental.pallas import tpu_sc as plsc

N = 10000
E = 320000
R = 8
B = 4
D = 128

NC = 2
NS = 16
NW = NC * NS
C = 80
ROWS_PER_W = E // C // NW
NPASS = 5
PR = ROWS_PER_W // NPASS
NPAD = 10240
RPT = NPAD // NS
ZC = 64
CW = D

TN = 2000


def _dense_body(comp_ref, x_ref, basis_ref, root_ref, h_ref, xr_ref):
    x = x_ref[...]
    for r in range(R):
        w = comp_ref[r, 0] * basis_ref[0]
        for b in range(1, B):
            w = w + comp_ref[r, b] * basis_ref[b]
        h_ref[r] = jnp.dot(x, w, preferred_element_type=jnp.float32)
    xr_ref[...] = jnp.dot(x, root_ref[...], preferred_element_type=jnp.float32)


_dense = pl.pallas_call(
    _dense_body,
    grid=(N // TN,),
    in_specs=[
        pl.BlockSpec(memory_space=pltpu.SMEM),
        pl.BlockSpec((TN, D), lambda i: (i, 0)),
        pl.BlockSpec((B, D, D), lambda i: (0, 0, 0)),
        pl.BlockSpec((D, D), lambda i: (0, 0)),
    ],
    out_specs=[
        pl.BlockSpec((R, TN, D), lambda i: (0, i, 0)),
        pl.BlockSpec((TN, D), lambda i: (i, 0)),
    ],
    out_shape=[
        jax.ShapeDtypeStruct((R, N, D), jnp.float32),
        jax.ShapeDtypeStruct((N, D), jnp.float32),
    ],
)


def _combine_body(p_ref, cnt_ref, xr_ref, bias_ref, o_ref):
    agg = p_ref[0] + p_ref[1]
    deg = cnt_ref[0, :, 0:1] + cnt_ref[1, :, 0:1]
    inv = 1.0 / jnp.maximum(deg, 1.0)
    v = agg * inv + xr_ref[...] + bias_ref[...]
    o_ref[...] = jnp.where(v > 0, v, jnp.exp(jnp.minimum(v, 0.0)) - 1.0)


_combine = pl.pallas_call(
    _combine_body,
    grid=(N // TN,),
    in_specs=[
        pl.BlockSpec((NC, TN, D), lambda i: (0, i, 0)),
        pl.BlockSpec((NC, TN, CW), lambda i: (0, i, 0)),
        pl.BlockSpec((TN, D), lambda i: (i, 0)),
        pl.BlockSpec((1, D), lambda i: (0, 0)),
    ],
    out_specs=pl.BlockSpec((TN, D), lambda i: (i, 0)),
    out_shape=jax.ShapeDtypeStruct((N, D), jnp.float32),
)


_mesh = plsc.VectorSubcoreMesh(core_axis_name="c", subcore_axis_name="s")


@functools.partial(
    pl.kernel,
    out_type=jax.ShapeDtypeStruct((NC, NPAD, D), jnp.float32),
    mesh=_mesh,
    scratch_types=[
        pltpu.VMEM((PR, C), jnp.int32),
        pltpu.VMEM((PR, C), jnp.int32),
        pltpu.VMEM((C, D), jnp.float32),
        pltpu.VMEM((C, D), jnp.float32),
        pltpu.VMEM((C, D), jnp.float32),
        pltpu.VMEM_SHARED((NPAD, D), jnp.float32),
        pltpu.SemaphoreType.DMA,
        pltpu.SemaphoreType.DMA,
        pltpu.SemaphoreType.DMA,
        pltpu.SemaphoreType.DMA,
        pltpu.SemaphoreType.DMA,
        pltpu.SemaphoreType.DMA,
        pltpu.SemaphoreType.DMA,
        pltpu.SemaphoreType.DMA,
    ],
)
def _sc_agg(h_hbm, gidx_hbm, didx_hbm, out_hbm,
            gbuf, dbuf, rows0, rows1, rows2, acc,
            sg0, sg1, sg2, ss0, ss1, ss2, si0, si1):
    cid = lax.axis_index("c")
    sid = lax.axis_index("s")
    wid = sid * NC + cid

    bufs = (rows0, rows1, rows2)
    sgs = (sg0, sg1, sg2)
    sss = (ss0, ss1, ss2)

    pltpu.async_copy(gidx_hbm.at[wid, 0], gbuf, si0)
    pltpu.async_copy(didx_hbm.at[wid, 0], dbuf, si1)

    def zrow(i, carry):
        for cc in range(D // 16):
            rows0[i, pl.ds(cc * 16, 16)] = jnp.zeros((16,), jnp.float32)
        return carry

    lax.fori_loop(0, ZC, zrow, 0)
    for k in range(RPT // ZC):
        pltpu.sync_copy(rows0.at[pl.ds(0, ZC)],
                        acc.at[pl.ds(sid * RPT + k * ZC, ZC)])

    pltpu.make_async_copy(gidx_hbm.at[wid, 0], gbuf, si0).wait()
    pltpu.make_async_copy(didx_hbm.at[wid, 0], dbuf, si1).wait()
    plsc.subcore_barrier()

    def g_start(j, b):
        pltpu.async_copy(h_hbm.at[gbuf.at[j]], bufs[b], sgs[b])

    def g_wait(j, b):
        pltpu.make_async_copy(h_hbm.at[gbuf.at[j]], bufs[b], sgs[b]).wait()

    def s_start(j, b):
        pltpu.async_copy(bufs[b], acc.at[dbuf.at[j]], sss[b], add=True)

    def s_wait(j, b):
        pltpu.make_async_copy(bufs[b], acc.at[dbuf.at[j]], sss[b]).wait()

    for p in range(NPASS):
        if p > 0:
            pltpu.sync_copy(gidx_hbm.at[wid, p], gbuf)
            pltpu.sync_copy(didx_hbm.at[wid, p], dbuf)
        g_start(0, 0)
        g_start(1, 1)
        g_wait(0, 0)
        s_start(0, 0)
        g_start(2, 2)
        g_wait(1, 1)
        s_start(1, 1)
        s_wait(0, 0)
        g_start(3, 0)

        def block(m, carry):
            j = 3 * m + 2
            for i in range(3):
                b = (2 + i) % 3
                g_wait(j + i, b)
                s_start(j + i, b)
                s_wait(j + i - 1, (b + 2) % 3)
                g_start(j + i + 2, (b + 2) % 3)
            return carry

        lax.fori_loop(0, (PR - 4) // 3, block, 0)
        ja = PR - 2
        jb = PR - 1
        g_wait(ja, ja % 3)
        s_start(ja, ja % 3)
        s_wait(ja - 1, (ja - 1) % 3)
        g_wait(jb, jb % 3)
        s_start(jb, jb % 3)
        s_wait(ja, ja % 3)
        s_wait(jb, jb % 3)

    plsc.subcore_barrier()
    pltpu.sync_copy(acc.at[pl.ds(sid * RPT, RPT)],
                    out_hbm.at[cid, pl.ds(sid * RPT, RPT)])


@functools.partial(
    pl.kernel,
    out_type=jax.ShapeDtypeStruct((NC, NPAD, CW), jnp.float32),
    mesh=_mesh,
    scratch_types=[
        pltpu.VMEM((PR, C), jnp.int32),
        pltpu.VMEM((C, CW), jnp.float32),
        pltpu.VMEM_SHARED((NPAD, CW), jnp.float32),
        pltpu.SemaphoreType.DMA,
        pltpu.SemaphoreType.DMA,
    ],
)
def _sc_deg(didx_hbm, out_hbm, dbuf, ones, cnt, sem0, sem1):
    cid = lax.axis_index("c")
    sid = lax.axis_index("s")
    wid = sid * NC + cid

    sems = (sem0, sem1)

    def zrow(i, carry):
        for cc in range(CW // 16):
            ones[i, pl.ds(cc * 16, 16)] = jnp.zeros((16,), jnp.float32)
        return carry

    lax.fori_loop(0, ZC, zrow, 0)
    for k in range(RPT // ZC):
        pltpu.sync_copy(ones.at[pl.ds(0, ZC)],
                        cnt.at[pl.ds(sid * RPT + k * ZC, ZC)])

    def orow(i, carry):
        for cc in range(CW // 16):
            ones[i, pl.ds(cc * 16, 16)] = jnp.ones((16,), jnp.float32)
        return carry

    lax.fori_loop(0, C, orow, 0)
    plsc.subcore_barrier()

    def s_start(j, b):
        pltpu.async_copy(ones, cnt.at[dbuf.at[j]], sems[b], add=True)

    def s_wait(j, b):
        pltpu.make_async_copy(ones, cnt.at[dbuf.at[j]], sems[b]).wait()

    for p in range(NPASS):
        pltpu.sync_copy(didx_hbm.at[wid, p], dbuf)
        s_start(0, 0)
        s_start(1, 1)

        def step(m, carry):
            j = 2 * m + 2
            s_wait(j - 2, 0)
            s_start(j, 0)
            s_wait(j - 1, 1)
            s_start(j + 1, 1)
            return carry

        lax.fori_loop(0, (PR - 3) // 2, step, 0)
        jl = PR - 1
        s_wait(jl - 2, (jl - 2) % 2)
        s_start(jl, jl % 2)
        s_wait(jl - 1, (jl - 1) % 2)
        s_wait(jl, jl % 2)
    plsc.subcore_barrier()
    pltpu.sync_copy(cnt.at[pl.ds(sid * RPT, RPT)],
                    out_hbm.at[cid, pl.ds(sid * RPT, RPT)])


def kernel(x, edge_index, edge_attr,
           basis0, comp0, root0, bias0,
           basis1, comp1, root1, bias1,
           basis2, comp2, root2, bias2,
           basis3, comp3, root3, bias3):
    src = edge_index[0]
    dst = edge_index[1]
    gidx = (edge_attr * N + src).astype(jnp.int32).reshape(NW, NPASS, PR, C)
    didx = dst.astype(jnp.int32).reshape(NW, NPASS, PR, C)

    cnt = _sc_deg(didx)

    h = x
    for basis, comp, root, bias in (
        (basis0, comp0, root0, bias0),
        (basis1, comp1, root1, bias1),
        (basis2, comp2, root2, bias2),
        (basis3, comp3, root3, bias3),
    ):
        hr, xr = _dense(comp, h, basis, root)
        part = _sc_agg(hr.reshape(R * N, D), gidx, didx)
        h = _combine(part, cnt, xr, bias.reshape(1, D))
    return h
